# baseline (speedup 1.0000x reference)
import jax
import jax.numpy as jnp
from jax.experimental import pallas as pl
from jax.experimental.pallas import tpu as pltpu

_BM = 400
_HALF = _BM // 2


def _mm_kernel(adj_a_ref, adj_b_ref, h_ref, out_ref):
    h = h_ref[...]
    out_ref[pl.ds(0, _HALF), :] = jnp.dot(
        adj_a_ref[...], h, preferred_element_type=jnp.float32)
    out_ref[pl.ds(_HALF, _HALF), :] = jnp.dot(
        adj_b_ref[...], h, preferred_element_type=jnp.float32)


def kernel(adj, h):
    n, k = adj.shape
    d = h.shape[1]
    grid = (n // _BM,)
    return pl.pallas_call(
        _mm_kernel,
        grid=grid,
        in_specs=[
            pl.BlockSpec((_HALF, k), lambda i: (2 * i, 0)),
            pl.BlockSpec((_HALF, k), lambda i: (2 * i + 1, 0)),
            pl.BlockSpec((k, d), lambda i: (0, 0),
                         pipeline_mode=pl.Buffered(buffer_count=1)),
        ],
        out_specs=pl.BlockSpec((_BM, d), lambda i: (i, 0)),
        out_shape=jax.ShapeDtypeStruct((n, d), jnp.float32),
        compiler_params=pltpu.CompilerParams(
            dimension_semantics=("parallel",)),
    )(adj, adj, h)
